# Initial kernel scaffold; baseline (speedup 1.0000x reference)
import functools

import jax
import jax.numpy as jnp
from jax import lax
from jax.experimental import pallas as pl
from jax.experimental.pallas import tpu as pltpu
from jax.experimental.pallas import tpu_sc as plsc

B = 16384
K = 64
NC = 2
NS = 16
NW = NC * NS
BPW = B // NW
CH = BPW // 128
L = 16
KC = K // L

_mesh = plsc.VectorSubcoreMesh(core_axis_name="c", subcore_axis_name="s")


@functools.partial(
    pl.kernel,
    out_type=jax.ShapeDtypeStruct((B,), jnp.float32),
    mesh=_mesh,
    scratch_types=[
        pltpu.VMEM((3, CH, 128), jnp.int32),
        pltpu.VMEM((BPW, K), jnp.float32),
        pltpu.VMEM((BPW, K), jnp.float32),
        pltpu.VMEM((BPW, K), jnp.float32),
        pltpu.VMEM((L * BPW,), jnp.float32),
        pltpu.VMEM((BPW,), jnp.float32),
        pltpu.SemaphoreType.DMA,
    ],
)
def _parafac_sc(idx_hbm, f0_hbm, f1_hbm, f2_hbm, out_hbm,
                idx_v, r0, r1, r2, st, outv, sem):
    wid = lax.axis_index("s") * NC + lax.axis_index("c")

    pltpu.sync_copy(idx_hbm.at[wid], idx_v)

    copies = []
    for t, (tab, r) in enumerate(((f0_hbm, r0), (f1_hbm, r1), (f2_hbm, r2))):
        for j in range(CH):
            copies.append(
                pltpu.async_copy(tab.at[idx_v.at[t, j]],
                                 r.at[pl.ds(j * 128, 128)], sem))
    for cp in copies:
        cp.wait()

    lane_stride = lax.iota(jnp.int32, L) * BPW
    cols = [lax.iota(jnp.int32, L) + c * L for c in range(KC)]

    def row_body(b, carry):
        rb = jnp.full((L,), b, jnp.int32)
        acc = None
        for c in range(KC):
            g0 = plsc.load_gather(r0, [rb, cols[c]])
            g1 = plsc.load_gather(r1, [rb, cols[c]])
            g2 = plsc.load_gather(r2, [rb, cols[c]])
            p = g0 * g1 * g2
            acc = p if acc is None else acc + p
        plsc.store_scatter(st, [lane_stride + b], acc)
        return carry

    lax.fori_loop(0, BPW, row_body, 0)

    def red_body(g, carry):
        b0 = g * L
        acc = st[pl.ds(b0, L)]
        for lane in range(1, L):
            acc = acc + st[pl.ds(lane * BPW + b0, L)]
        outv[pl.ds(b0, L)] = acc
        return carry

    lax.fori_loop(0, BPW // L, red_body, 0)

    pltpu.sync_copy(outv, out_hbm.at[pl.ds(wid * BPW, BPW)])


def kernel(indices, f0, f1, f2):
    out_dtype = f0.dtype
    idx = indices.astype(jnp.int32).reshape(3, NW, CH, 128).transpose(1, 0, 2, 3)
    out = _parafac_sc(idx,
                      f0.astype(jnp.float32),
                      f1.astype(jnp.float32),
                      f2.astype(jnp.float32))
    return out.astype(out_dtype)


# speedup vs baseline: 1.7119x; 1.7119x over previous
import functools

import jax
import jax.numpy as jnp
from jax import lax
from jax.experimental import pallas as pl
from jax.experimental.pallas import tpu as pltpu
from jax.experimental.pallas import tpu_sc as plsc

B = 16384
K = 64
NC = 2
NS = 16
NW = NC * NS
BPW = B // NW
CH = BPW // 128
L = 16
KC = K // L

_mesh = plsc.VectorSubcoreMesh(core_axis_name="c", subcore_axis_name="s",
                               num_cores=NC, num_subcores=NS)


@functools.partial(
    pl.kernel,
    out_type=jax.ShapeDtypeStruct((B,), jnp.float32),
    mesh=_mesh,
    compiler_params=pltpu.CompilerParams(needs_layout_passes=False,
                                         use_tc_tiling_on_sc=False),
    scratch_types=[
        pltpu.VMEM((3, CH, 128), jnp.int32),
        pltpu.VMEM((BPW, K), jnp.float32),
        pltpu.VMEM((BPW, K), jnp.float32),
        pltpu.VMEM((BPW, K), jnp.float32),
        pltpu.VMEM((L * BPW,), jnp.float32),
        pltpu.VMEM((BPW,), jnp.float32),
        pltpu.SemaphoreType.DMA,
    ],
)
def _parafac_sc(idx_hbm, f0_hbm, f1_hbm, f2_hbm, out_hbm,
                idx_v, r0, r1, r2, st, outv, sem):
    wid = lax.axis_index("s") * NC + lax.axis_index("c")

    pltpu.sync_copy(idx_hbm.at[wid], idx_v)

    copies = []
    for t, (tab, r) in enumerate(((f0_hbm, r0), (f1_hbm, r1), (f2_hbm, r2))):
        for j in range(CH):
            copies.append(
                pltpu.async_copy(tab.at[idx_v.at[jnp.int32(t), jnp.int32(j)]],
                                 r.at[pl.ds(j * 128, 128)], sem))
    for cp in copies:
        cp.wait()

    lane_stride = lax.iota(jnp.int32, L) * BPW
    cols = [lax.iota(jnp.int32, L) + c * L for c in range(KC)]

    def row_body(b, carry):
        rb = jnp.full((L,), b, jnp.int32)
        acc = None
        for c in range(KC):
            g0 = plsc.load_gather(r0, [rb, cols[c]])
            g1 = plsc.load_gather(r1, [rb, cols[c]])
            g2 = plsc.load_gather(r2, [rb, cols[c]])
            p = g0 * g1 * g2
            acc = p if acc is None else acc + p
        plsc.store_scatter(st, [lane_stride + b], acc)
        return carry

    lax.fori_loop(jnp.int32(0), jnp.int32(BPW), row_body, jnp.int32(0))

    def red_body(g, carry):
        b0 = g * L
        acc = st[pl.ds(b0, L)]
        for lane in range(1, L):
            acc = acc + st[pl.ds(lane * BPW + b0, L)]
        outv[pl.ds(b0, L)] = acc
        return carry

    lax.fori_loop(jnp.int32(0), jnp.int32(BPW // L), red_body, jnp.int32(0))

    pltpu.sync_copy(outv, out_hbm.at[pl.ds(wid * BPW, BPW)])


def kernel(indices, f0, f1, f2):
    out_dtype = f0.dtype
    idx = indices.astype(jnp.int32).reshape(3, NW, CH, 128).transpose(1, 0, 2, 3)
    out = _parafac_sc(idx,
                      f0.astype(jnp.float32),
                      f1.astype(jnp.float32),
                      f2.astype(jnp.float32))
    return out.astype(out_dtype)
